# baseline (speedup 1.0000x reference)
import dataclasses

import jax
import jax.numpy as jnp
from jax.experimental import pallas as pl
from jax.experimental.pallas import tpu as pltpu
from jax.experimental.pallas import tpu_sc as plsc

_B = 16
_T = 576
_E = 64
_N = 1025
_NE = 1024
_LEGACY = 0.2
_TILE = 512


def _c1_body(ks_ref, wt_ref, w32_ref, minidx_ref):
    ks = ks_ref[...]
    wt = wt_ref[...]
    rowssq = jnp.sum(ks * ks, axis=1, keepdims=True)
    wsq = jnp.sum(wt * wt, axis=0)
    mm = jax.lax.dot_general(ks, wt, (((1,), (0,)), ((), ())),
                             preferred_element_type=jnp.float32)
    d = rowssq + wsq[None, :] - 2.0 * mm
    dmin = jnp.min(d, axis=1, keepdims=True)
    ii = jax.lax.broadcasted_iota(jnp.int32, d.shape, 1)
    minidx = jnp.min(jnp.where(d == dmin, ii, jnp.int32(2**30)), axis=1)
    minidx_ref[...] = minidx[:, None]
    advb = (d[:, :_NE] > d[:, 1:_N]) & (ii[:, :_NE] < (_NE - 1))
    advf = advb.astype(jnp.float32)
    jrow = jax.lax.broadcasted_iota(jnp.int32, (_NE, 32), 0)
    kcol = jax.lax.broadcasted_iota(jnp.int32, (_NE, 32), 1)
    bitpos = jrow & 31
    hit = (jrow >> 5) == kcol
    in_lo = bitpos < 16
    p_lo = jnp.where(hit & in_lo,
                     (1 << jnp.where(in_lo, bitpos, 0)).astype(jnp.float32),
                     0.0)
    p_hi = jnp.where(hit & (~in_lo),
                     (1 << jnp.maximum(bitpos - 16, 0)).astype(jnp.float32),
                     0.0)
    dn = (((1,), (0,)), ((), ()))
    lo = jax.lax.dot_general(advf, p_lo, dn,
                             preferred_element_type=jnp.float32)
    hi = jax.lax.dot_general(advf, p_hi, dn,
                             preferred_element_type=jnp.float32)
    w32_ref[...] = lo.astype(jnp.int32) | (hi.astype(jnp.int32) << 16)


def _call1(ksf, wt):
    n_tiles = ksf.shape[0] // _TILE
    return pl.pallas_call(
        _c1_body,
        grid=(n_tiles,),
        in_specs=[pl.BlockSpec((_TILE, _E), lambda i: (i, 0)),
                  pl.BlockSpec((_E, _N), lambda i: (0, 0))],
        out_specs=[pl.BlockSpec((_TILE, 32), lambda i: (i, 0)),
                   pl.BlockSpec((_TILE, 1), lambda i: (i, 0))],
        out_shape=[jax.ShapeDtypeStruct((_T * _B, 32), jnp.int32),
                   jax.ShapeDtypeStruct((_T * _B, 1), jnp.int32)],
    )(ksf, wt)


def _call2_sc(w32_bm, mi_bm, w):
    mesh = plsc.VectorSubcoreMesh(core_axis_name="c", subcore_axis_name="s",
                                  num_cores=2, num_subcores=16)
    cp = pltpu.CompilerParams()
    if "needs_layout_passes" in pltpu.CompilerParams.__dataclass_fields__:
        cp = dataclasses.replace(cp, needs_layout_passes=False)

    @pl.kernel(
        out_type=[jax.ShapeDtypeStruct((_B, _T), jnp.int32),
                  jax.ShapeDtypeStruct((_B, _T, 2 * _E), jnp.float32),
                  jax.ShapeDtypeStruct((_B, _T, 2 * _E), jnp.float32),
                  jax.ShapeDtypeStruct((_B, _T, 2 * _E), jnp.float32)],
        mesh=mesh,
        compiler_params=cp,
        scratch_types=[pltpu.VMEM((_T * 32,), jnp.int32),
                       pltpu.VMEM((_T,), jnp.int32),
                       pltpu.VMEM((_T * 16,), jnp.int32),
                       pltpu.VMEM((_T,), jnp.int32),
                       pltpu.VMEM((_T,), jnp.int32),
                       pltpu.VMEM((_T, 2 * _E), jnp.float32),
                       pltpu.SemaphoreType.DMA,
                       pltpu.SemaphoreType.DMA,
                       pltpu.SemaphoreType.DMA,
                       pltpu.SemaphoreType.DMA],
    )
    def scan_kernel(w32_ref, mi_ref, w_ref, enc_ref, khh_ref, khn_ref,
                    kmin_ref, words, miv, encl, encc, encn, bufa,
                    s0, s1, s2, s3):
        c = jax.lax.axis_index("c")
        s = jax.lax.axis_index("s")
        b = c * (_B // 2) + s

        @pl.when(s < (_B // 2))
        def _():
            cw = pltpu.async_copy(w32_ref.at[b], words, s0)
            cm = pltpu.async_copy(mi_ref.at[b], miv, s1)
            cw.wait()
            cm.wait()
            zero16 = jnp.zeros((16,), jnp.int32)
            ind0 = jnp.minimum(plsc.load_gather(miv, [zero16]), _NE - 1)
            encl[pl.ds(0, 16)] = ind0

            def step(t, ind):
                w_ = plsc.load_gather(words, [t * 32 + (ind >> 5)])
                ind = ind + ((w_ >> (ind & 31)) & 1)
                encl[pl.ds(t * 16, 16)] = ind
                return ind

            jax.lax.fori_loop(1, _T, step, ind0)

            lane16 = jax.lax.iota(jnp.int32, 16)

            def compact(i, _):
                v = plsc.load_gather(encl, [lane16 * 16 + i * 256])
                encc[pl.ds(i * 16, 16)] = v
                encn[pl.ds(i * 16, 16)] = jnp.minimum(v + 1, _NE - 1)
                return 0

            jax.lax.fori_loop(0, _T // 16, compact, 0)

            ce = pltpu.async_copy(encc, enc_ref.at[b], s2)
            g0 = pltpu.async_copy(w_ref.at[encc], bufa, s0)
            g0.wait()
            c0 = pltpu.async_copy(bufa, khh_ref.at[b], s3)
            c0.wait()
            g1 = pltpu.async_copy(w_ref.at[encn], bufa, s1)
            g1.wait()
            c1 = pltpu.async_copy(bufa, khn_ref.at[b], s3)
            c1.wait()
            g2 = pltpu.async_copy(w_ref.at[miv], bufa, s0)
            g2.wait()
            c2 = pltpu.async_copy(bufa, kmin_ref.at[b], s0)
            ce.wait()
            c2.wait()

    return scan_kernel(w32_bm, mi_bm, w)


def _c3_body(ks_ref, khh_ref, khn_ref, kmin_ref, enc_ref,
             kh_ref, lh_ref, ln_ref, v_ref, em_ref, led_ref):
    i = pl.program_id(0)
    ks = ks_ref[...]
    khh = khh_ref[...][:, :_E]
    khn = khn_ref[...][:, :_E]
    kmin = kmin_ref[...][:, :_E]
    dh = ks - khh
    s_here = jnp.sum(dh * dh, axis=1)
    dnx = ks - khn
    s_next = jnp.sum(dnx * dnx, axis=1)
    dm = ks - kmin
    s_min = jnp.sum(dm * dm, axis=1)
    base_h = s_here + s_here * _LEGACY
    base_n = s_next + s_next * _LEGACY
    lmi = s_min + s_min * _LEGACY
    lm_h = jnp.where(lmi < base_h, lmi, 0.0)
    lm_n = jnp.where(lmi < base_n, lmi, 0.0)
    dd = s_next - s_here
    en = dd + dd * _LEGACY
    kh_ref[...] = ks + (khh - ks)
    lh_ref[...] = (base_h + (-base_n) - lm_h)[:, None]
    ln_ref[...] = (base_n + (-base_h) - lm_n)[:, None]

    enc = enc_ref[...][:, 0]
    change = (enc[1:] - enc[:-1]) != 0
    ec = jnp.where(change, 0.0, en[1:] - en[:-1])
    led_part = jnp.sum(jnp.maximum(ec + (1e-06 / _NE), 0.0))
    em_part = jnp.sum(en)
    v_part = jnp.max(enc) - jnp.min(enc)

    @pl.when(i == 0)
    def _():
        v_ref[...] = jnp.zeros((1, 1), jnp.int32)
        em_ref[...] = jnp.zeros((1, 1), jnp.float32)
        led_ref[...] = jnp.zeros((1, 1), jnp.float32)

    v_ref[...] = jnp.maximum(v_ref[...], jnp.reshape(v_part, (1, 1)))
    em_ref[...] = em_ref[...] + jnp.reshape(em_part, (1, 1))
    led_ref[...] = led_ref[...] + jnp.reshape(led_part, (1, 1))

    @pl.when(i == _B - 1)
    def _():
        em_ref[...] = em_ref[...] / (_B * _T)
        led_ref[...] = led_ref[...] / (_B * (_T - 1))


def _call3(ksf, khh, khn, kmin, enc):
    return pl.pallas_call(
        _c3_body,
        grid=(_B,),
        in_specs=[pl.BlockSpec((_T, _E), lambda i: (i, 0)),
                  pl.BlockSpec((_T, 2 * _E), lambda i: (i, 0)),
                  pl.BlockSpec((_T, 2 * _E), lambda i: (i, 0)),
                  pl.BlockSpec((_T, 2 * _E), lambda i: (i, 0)),
                  pl.BlockSpec((_T, 1), lambda i: (i, 0))],
        out_specs=[pl.BlockSpec((_T, _E), lambda i: (i, 0)),
                   pl.BlockSpec((_T, 1), lambda i: (i, 0)),
                   pl.BlockSpec((_T, 1), lambda i: (i, 0)),
                   pl.BlockSpec((1, 1), lambda i: (0, 0)),
                   pl.BlockSpec((1, 1), lambda i: (0, 0)),
                   pl.BlockSpec((1, 1), lambda i: (0, 0))],
        out_shape=[jax.ShapeDtypeStruct((_T * _B, _E), jnp.float32),
                   jax.ShapeDtypeStruct((_T * _B, 1), jnp.float32),
                   jax.ShapeDtypeStruct((_T * _B, 1), jnp.float32),
                   jax.ShapeDtypeStruct((1, 1), jnp.int32),
                   jax.ShapeDtypeStruct((1, 1), jnp.float32),
                   jax.ShapeDtypeStruct((1, 1), jnp.float32)],
    )(ksf, khh, khn, kmin, enc)


def kernel(key_soft, W):
    B, T, E = key_soft.shape
    ksf = key_soft.reshape(B * T, E)
    wt = W.T
    w32, minidx = _call1(ksf, wt)
    w32_bm = w32.reshape(B, T * 32)
    mi_bm = minidx.reshape(B, T)
    w128 = jnp.concatenate([W, jnp.zeros_like(W)], axis=1)
    enc_bm, khh, khn, kmin = _call2_sc(w32_bm, mi_bm, w128)
    enc = enc_bm.reshape(B * T, 1)
    kh, lh, ln, v, em, led = _call3(ksf, khh.reshape(B * T, 2 * E),
                                    khn.reshape(B * T, 2 * E),
                                    kmin.reshape(B * T, 2 * E), enc)
    return (kh.reshape(B, T, E), enc_bm, v[0, 0],
            lh.reshape(B, T), ln.reshape(B, T), em[0, 0], led[0, 0])


# speedup vs baseline: 2.3529x; 2.3529x over previous
import dataclasses

import jax
import jax.numpy as jnp
from jax.experimental import pallas as pl
from jax.experimental.pallas import tpu as pltpu
from jax.experimental.pallas import tpu_sc as plsc

_B = 16
_T = 576
_E = 64
_N = 1025
_NE = 1024
_LEGACY = 0.2
_TILE = 512


def _c1_body(ks_ref, wt_ref, w32_ref, minidx_ref):
    ks = ks_ref[...]
    wt = wt_ref[...]
    rowssq = jnp.sum(ks * ks, axis=1, keepdims=True)
    wsq = jnp.sum(wt * wt, axis=0)
    mm = jax.lax.dot_general(ks, wt, (((1,), (0,)), ((), ())),
                             preferred_element_type=jnp.float32)
    d = rowssq + wsq[None, :] - 2.0 * mm
    dmin = jnp.min(d, axis=1, keepdims=True)
    ii = jax.lax.broadcasted_iota(jnp.int32, d.shape, 1)
    minidx = jnp.min(jnp.where(d == dmin, ii, jnp.int32(2**30)), axis=1)
    minidx_ref[...] = minidx[:, None]
    advb = (d[:, :_NE] > d[:, 1:_N]) & (ii[:, :_NE] < (_NE - 1))
    advf = advb.astype(jnp.float32)
    jrow = jax.lax.broadcasted_iota(jnp.int32, (_NE, 32), 0)
    kcol = jax.lax.broadcasted_iota(jnp.int32, (_NE, 32), 1)
    bitpos = jrow & 31
    hit = (jrow >> 5) == kcol
    in_lo = bitpos < 16
    p_lo = jnp.where(hit & in_lo,
                     (1 << jnp.where(in_lo, bitpos, 0)).astype(jnp.float32),
                     0.0)
    p_hi = jnp.where(hit & (~in_lo),
                     (1 << jnp.maximum(bitpos - 16, 0)).astype(jnp.float32),
                     0.0)
    dn = (((1,), (0,)), ((), ()))
    lo = jax.lax.dot_general(advf, p_lo, dn,
                             preferred_element_type=jnp.float32)
    hi = jax.lax.dot_general(advf, p_hi, dn,
                             preferred_element_type=jnp.float32)
    w32_ref[...] = lo.astype(jnp.int32) | (hi.astype(jnp.int32) << 16)


def _call1(ksf, wt):
    n_tiles = ksf.shape[0] // _TILE
    return pl.pallas_call(
        _c1_body,
        grid=(n_tiles,),
        in_specs=[pl.BlockSpec((_TILE, _E), lambda i: (i, 0)),
                  pl.BlockSpec((_E, _N), lambda i: (0, 0))],
        out_specs=[pl.BlockSpec((_TILE, 32), lambda i: (i, 0)),
                   pl.BlockSpec((_TILE, 1), lambda i: (i, 0))],
        out_shape=[jax.ShapeDtypeStruct((_T * _B, 32), jnp.int32),
                   jax.ShapeDtypeStruct((_T * _B, 1), jnp.int32)],
    )(ksf, wt)


def _call2_sc(w32_bm, mi_bm):
    mesh = plsc.VectorSubcoreMesh(core_axis_name="c", subcore_axis_name="s",
                                  num_cores=2, num_subcores=16)
    cp = pltpu.CompilerParams()
    if "needs_layout_passes" in pltpu.CompilerParams.__dataclass_fields__:
        cp = dataclasses.replace(cp, needs_layout_passes=False)

    @pl.kernel(
        out_type=jax.ShapeDtypeStruct((_B, _T), jnp.int32),
        mesh=mesh,
        compiler_params=cp,
        scratch_types=[pltpu.VMEM((_T * 32,), jnp.int32),
                       pltpu.VMEM((_T,), jnp.int32),
                       pltpu.VMEM((_T * 16,), jnp.int32),
                       pltpu.VMEM((_T,), jnp.int32),
                       pltpu.SemaphoreType.DMA,
                       pltpu.SemaphoreType.DMA],
    )
    def scan_kernel(w32_ref, mi_ref, enc_ref, words, miv, encl, encc,
                    s0, s1):
        c = jax.lax.axis_index("c")
        s = jax.lax.axis_index("s")
        b = c * (_B // 2) + s

        @pl.when(s < (_B // 2))
        def _():
            cw = pltpu.async_copy(w32_ref.at[b], words, s0)
            cm = pltpu.async_copy(mi_ref.at[b], miv, s1)
            cw.wait()
            cm.wait()
            zero16 = jnp.zeros((16,), jnp.int32)
            ind0 = jnp.minimum(plsc.load_gather(miv, [zero16]), _NE - 1)
            encl[pl.ds(0, 16)] = ind0

            def step(t, ind):
                w_ = plsc.load_gather(words, [t * 32 + (ind >> 5)])
                ind = ind + ((w_ >> (ind & 31)) & 1)
                encl[pl.ds(t * 16, 16)] = ind
                return ind

            jax.lax.fori_loop(1, _T, step, ind0)

            lane16 = jax.lax.iota(jnp.int32, 16)

            def compact(i, _):
                v = plsc.load_gather(encl, [lane16 * 16 + i * 256])
                encc[pl.ds(i * 16, 16)] = v
                return 0

            jax.lax.fori_loop(0, _T // 16, compact, 0)
            pltpu.async_copy(encc, enc_ref.at[b], s0).wait()

    return scan_kernel(w32_bm, mi_bm)


def _c3_body(ks_ref, w_ref, enc_ref, mi_ref,
             kh_ref, lh_ref, ln_ref, v_ref, em_ref, led_ref):
    i = pl.program_id(0)
    ks = ks_ref[...]
    w = w_ref[...]
    ind = enc_ref[...]
    indn = jnp.minimum(ind + 1, _NE - 1)
    mi = mi_ref[...]
    jj = jax.lax.broadcasted_iota(jnp.int32, (_T, _N), 1)
    dnum = (((1,), (0,)), ((), ()))
    khh = jax.lax.dot_general((jj == ind).astype(jnp.float32), w, dnum,
                              preferred_element_type=jnp.float32)
    khn = jax.lax.dot_general((jj == indn).astype(jnp.float32), w, dnum,
                              preferred_element_type=jnp.float32)
    kmin = jax.lax.dot_general((jj == mi).astype(jnp.float32), w, dnum,
                               preferred_element_type=jnp.float32)
    dh = ks - khh
    s_here = jnp.sum(dh * dh, axis=1)
    dnx = ks - khn
    s_next = jnp.sum(dnx * dnx, axis=1)
    dm = ks - kmin
    s_min = jnp.sum(dm * dm, axis=1)
    base_h = s_here + s_here * _LEGACY
    base_n = s_next + s_next * _LEGACY
    lmi = s_min + s_min * _LEGACY
    lm_h = jnp.where(lmi < base_h, lmi, 0.0)
    lm_n = jnp.where(lmi < base_n, lmi, 0.0)
    dd = s_next - s_here
    en = dd + dd * _LEGACY
    kh_ref[...] = ks + (khh - ks)
    lh_ref[...] = (base_h + (-base_n) - lm_h)[:, None]
    ln_ref[...] = (base_n + (-base_h) - lm_n)[:, None]

    enc = ind[:, 0]
    change = (enc[1:] - enc[:-1]) != 0
    ec = jnp.where(change, 0.0, en[1:] - en[:-1])
    led_part = jnp.sum(jnp.maximum(ec + (1e-06 / _NE), 0.0))
    em_part = jnp.sum(en)
    v_part = jnp.max(enc) - jnp.min(enc)

    @pl.when(i == 0)
    def _():
        v_ref[...] = jnp.zeros((1, 1), jnp.int32)
        em_ref[...] = jnp.zeros((1, 1), jnp.float32)
        led_ref[...] = jnp.zeros((1, 1), jnp.float32)

    v_ref[...] = jnp.maximum(v_ref[...], jnp.reshape(v_part, (1, 1)))
    em_ref[...] = em_ref[...] + jnp.reshape(em_part, (1, 1))
    led_ref[...] = led_ref[...] + jnp.reshape(led_part, (1, 1))

    @pl.when(i == _B - 1)
    def _():
        em_ref[...] = em_ref[...] / (_B * _T)
        led_ref[...] = led_ref[...] / (_B * (_T - 1))


def _call3(ksf, w, enc, minidx):
    return pl.pallas_call(
        _c3_body,
        grid=(_B,),
        in_specs=[pl.BlockSpec((_T, _E), lambda i: (i, 0)),
                  pl.BlockSpec((_N, _E), lambda i: (0, 0)),
                  pl.BlockSpec((_T, 1), lambda i: (i, 0)),
                  pl.BlockSpec((_T, 1), lambda i: (i, 0))],
        out_specs=[pl.BlockSpec((_T, _E), lambda i: (i, 0)),
                   pl.BlockSpec((_T, 1), lambda i: (i, 0)),
                   pl.BlockSpec((_T, 1), lambda i: (i, 0)),
                   pl.BlockSpec((1, 1), lambda i: (0, 0)),
                   pl.BlockSpec((1, 1), lambda i: (0, 0)),
                   pl.BlockSpec((1, 1), lambda i: (0, 0))],
        out_shape=[jax.ShapeDtypeStruct((_T * _B, _E), jnp.float32),
                   jax.ShapeDtypeStruct((_T * _B, 1), jnp.float32),
                   jax.ShapeDtypeStruct((_T * _B, 1), jnp.float32),
                   jax.ShapeDtypeStruct((1, 1), jnp.int32),
                   jax.ShapeDtypeStruct((1, 1), jnp.float32),
                   jax.ShapeDtypeStruct((1, 1), jnp.float32)],
    )(ksf, w, enc, minidx)


def kernel(key_soft, W):
    B, T, E = key_soft.shape
    ksf = key_soft.reshape(B * T, E)
    wt = W.T
    w32, minidx = _call1(ksf, wt)
    w32_bm = w32.reshape(B, T * 32)
    mi_bm = minidx.reshape(B, T)
    enc_bm = _call2_sc(w32_bm, mi_bm)
    enc = enc_bm.reshape(B * T, 1)
    kh, lh, ln, v, em, led = _call3(ksf, W, enc, minidx)
    return (kh.reshape(B, T, E), enc_bm, v[0, 0],
            lh.reshape(B, T), ln.reshape(B, T), em[0, 0], led[0, 0])
